# baseline (speedup 1.0000x reference)
import functools

import jax
import jax.numpy as jnp
from jax import lax
from jax.experimental import pallas as pl
from jax.experimental.pallas import tpu as pltpu
from jax.experimental.pallas import tpu_sc as plsc


_R = 32


def kernel(x, embed_weight):
    B, S, D = x.shape
    NC, NS = 2, 16
    NW = NC * NS
    n_rows = B * S
    rows_per_w = n_rows // NW
    n_chunks = rows_per_w // _R
    xf = x.reshape(n_rows // _R, _R, D)
    eidx = (jnp.arange(n_rows, dtype=jnp.int32) % S).reshape(NW, n_chunks, _R)

    mesh = plsc.VectorSubcoreMesh(core_axis_name="c", subcore_axis_name="s")

    @functools.partial(
        pl.kernel,
        mesh=mesh,
        out_type=jax.ShapeDtypeStruct((n_rows // _R, _R, D), x.dtype),
        scratch_types=[
            pltpu.VMEM((_R, D), x.dtype),
            pltpu.VMEM((_R, D), x.dtype),
            pltpu.VMEM((n_chunks, _R), jnp.int32),
            pltpu.SemaphoreType.DMA,
            pltpu.SemaphoreType.DMA,
        ],
    )
    def sc_add(x_hbm, e_hbm, eidx_hbm, o_hbm, bufx, bufe, eiv, semx, seme):
        wid = lax.axis_index("s") * NC + lax.axis_index("c")
        pltpu.sync_copy(eidx_hbm.at[wid], eiv)
        for i in range(n_chunks):
            cid = wid * n_chunks + i
            cx = pltpu.make_async_copy(x_hbm.at[cid], bufx, semx)
            ce = pltpu.make_async_copy(e_hbm.at[eiv.at[i]], bufe, seme)
            cx.start()
            ce.start()
            cx.wait()
            ce.wait()

            def row_body(r):
                for k in range(D // 16):
                    bufx[r, pl.ds(16 * k, 16)] = (
                        bufx[r, pl.ds(16 * k, 16)] + bufe[r, pl.ds(16 * k, 16)])
            pl.loop(0, _R)(row_body)

            co = pltpu.make_async_copy(bufx, o_hbm.at[cid], semx)
            co.start()
            co.wait()

    out = sc_add(xf, embed_weight, eidx)
    return out.reshape(B, S, D)


# speedup vs baseline: 3.5027x; 3.5027x over previous
import jax
import jax.numpy as jnp
from jax.experimental import pallas as pl
from jax.experimental.pallas import tpu as pltpu


_CHUNK = 1024
_DEPTH = 5


def _add_kernel(x_hbm, e_hbm, o_hbm, xbuf, ebuf, obuf, xsem, esem, osem):
    n_chunks = x_hbm.shape[0]
    n_e = ebuf.shape[0]

    for j in range(n_e):
        pltpu.make_async_copy(e_hbm.at[j], ebuf.at[j], esem.at[j]).start()
    for c in range(_DEPTH):
        pltpu.make_async_copy(x_hbm.at[c], xbuf.at[c], xsem.at[c]).start()
    for j in range(n_e):
        pltpu.make_async_copy(e_hbm.at[j], ebuf.at[j], esem.at[j]).wait()

    for c in range(n_chunks):
        slot = c % _DEPTH
        pltpu.make_async_copy(x_hbm.at[c], xbuf.at[slot], xsem.at[slot]).wait()
        if c >= _DEPTH:
            pltpu.make_async_copy(
                obuf.at[slot], o_hbm.at[c - _DEPTH], osem.at[slot]).wait()
        obuf[slot] = xbuf[slot] + ebuf[c % n_e]
        pltpu.make_async_copy(obuf.at[slot], o_hbm.at[c], osem.at[slot]).start()
        if c + _DEPTH < n_chunks:
            pltpu.make_async_copy(
                x_hbm.at[c + _DEPTH], xbuf.at[slot], xsem.at[slot]).start()

    for c in range(n_chunks - _DEPTH, n_chunks):
        slot = c % _DEPTH
        pltpu.make_async_copy(obuf.at[slot], o_hbm.at[c], osem.at[slot]).wait()


def kernel(x, embed_weight):
    B, S, D = x.shape
    n_chunks = (B * S) // _CHUNK
    xf = x.reshape(n_chunks, _CHUNK, D)
    ef = embed_weight.reshape(embed_weight.shape[0] // _CHUNK, _CHUNK, D)
    n_e = S // _CHUNK
    out = pl.pallas_call(
        _add_kernel,
        in_specs=[
            pl.BlockSpec(memory_space=pl.ANY),
            pl.BlockSpec(memory_space=pl.ANY),
        ],
        out_specs=pl.BlockSpec(memory_space=pl.ANY),
        out_shape=jax.ShapeDtypeStruct((n_chunks, _CHUNK, D), x.dtype),
        scratch_shapes=[
            pltpu.VMEM((_DEPTH, _CHUNK, D), x.dtype),
            pltpu.VMEM((n_e, _CHUNK, D), x.dtype),
            pltpu.VMEM((_DEPTH, _CHUNK, D), x.dtype),
            pltpu.SemaphoreType.DMA((_DEPTH,)),
            pltpu.SemaphoreType.DMA((n_e,)),
            pltpu.SemaphoreType.DMA((_DEPTH,)),
        ],
    )(xf, ef)
    return out.reshape(B, S, D)
